# baseline (speedup 1.0000x reference)
import jax
import jax.numpy as jnp
from jax import lax
from jax.experimental import pallas as pl
from jax.experimental.pallas import tpu as pltpu
from jax.experimental.pallas import tpu_sc as plsc

_info = plsc.get_sparse_core_info()
_NC, _NS, _L = _info.num_cores, _info.num_subcores, _info.num_lanes
_NW = _NC * _NS
_B = 16384
_D = 64
_BPW = _B // _NW
_CHUNK = 128
_NCHUNK = _BPW // _CHUNK
_NG = _BPW // _L
_BITREV = [0, 8, 4, 12, 2, 10, 6, 14, 1, 9, 5, 13, 3, 11, 7, 15]


def _glove_body(cw_hbm, tw_hbm, cooc_hbm, wt_hbm, embv_hbm, embu_hbm, vb_hbm,
                out_hbm,
                cw_v, tw_v, cooc_v, wt_v, cemb, temb, cb_v, tb_v,
                rbuf, acc_v, sem):
    wid = lax.axis_index("s") * _NC + lax.axis_index("c")
    base = pl.multiple_of(wid * _BPW, _BPW)

    pltpu.sync_copy(cw_hbm.at[pl.ds(base, _BPW)], cw_v)
    pltpu.sync_copy(tw_hbm.at[pl.ds(base, _BPW)], tw_v)
    pltpu.sync_copy(cooc_hbm.at[pl.ds(base, _BPW)], cooc_v)
    pltpu.sync_copy(wt_hbm.at[pl.ds(base, _BPW)], wt_v)

    copies = []
    for c in range(_NCHUNK):
        s = pl.ds(c * _CHUNK, _CHUNK)
        copies.append(pltpu.async_copy(embv_hbm.at[cw_v.at[s]], cemb.at[s], sem))
        copies.append(pltpu.async_copy(embu_hbm.at[tw_v.at[s]], temb.at[s], sem))
        copies.append(pltpu.async_copy(vb_hbm.at[cw_v.at[s]], cb_v.at[s], sem))
        copies.append(pltpu.async_copy(vb_hbm.at[tw_v.at[s]], tb_v.at[s], sem))
    for cp in copies:
        cp.wait()

    lane = lax.iota(jnp.int32, _L)
    masks = {h: (lane & h) == 0 for h in (8, 4, 2, 1)}

    def group(g, acc):
        b0 = pl.multiple_of(g * _L, _L)
        vals = []
        for i, j in enumerate(_BITREV):
            b = b0 + j
            p = cemb[b, pl.ds(0, _L)] * temb[b, pl.ds(0, _L)]
            for k in range(1, _D // _L):
                p = p + cemb[b, pl.ds(k * _L, _L)] * temb[b, pl.ds(k * _L, _L)]
            base = 8 + 32 * i
            rbuf[pl.ds(base, _L)] = p
            vals.append((p, base))
        slot = _L
        for h in (8, 4, 2, 1):
            m = masks[h]
            nxt = []
            for t in range(len(vals) // 2):
                (av, ab), (bv, bb) = vals[2 * t], vals[2 * t + 1]
                a_rot = rbuf[pl.ds(ab + h, _L)]
                b_rot = rbuf[pl.ds(bb - h, _L)]
                c = jnp.where(m, av + a_rot, bv + b_rot)
                cb_ = -1
                if h > 1:
                    cb_ = 8 + 32 * slot
                    slot += 1
                    rbuf[pl.ds(cb_, _L)] = c
                nxt.append((c, cb_))
            vals = nxt
        dotv = vals[0][0]
        cb = cb_v[pl.ds(b0, _L)]
        tb = tb_v[pl.ds(b0, _L)]
        cooc = cooc_v[pl.ds(b0, _L)]
        wt = wt_v[pl.ds(b0, _L)]
        err = dotv + cb + tb - cooc
        return acc + wt * err * err

    acc = lax.fori_loop(0, _NG, group, jnp.zeros((_L,), jnp.float32))
    acc_v[...] = acc
    pltpu.sync_copy(acc_v, out_hbm.at[wid])


_glove_partials = pl.kernel(
    _glove_body,
    out_type=jax.ShapeDtypeStruct((_NW, _L), jnp.float32),
    mesh=plsc.VectorSubcoreMesh(core_axis_name="c", subcore_axis_name="s"),
    compiler_params=pltpu.CompilerParams(use_tc_tiling_on_sc=False),
    scratch_types=[
        pltpu.VMEM((_BPW,), jnp.int32),
        pltpu.VMEM((_BPW,), jnp.int32),
        pltpu.VMEM((_BPW,), jnp.float32),
        pltpu.VMEM((_BPW,), jnp.float32),
        pltpu.VMEM((_BPW, _D), jnp.float32),
        pltpu.VMEM((_BPW, _D), jnp.float32),
        pltpu.VMEM((_BPW,), jnp.float32),
        pltpu.VMEM((_BPW,), jnp.float32),
        pltpu.VMEM((1024,), jnp.float32),
        pltpu.VMEM((_L,), jnp.float32),
        pltpu.SemaphoreType.DMA,
    ],
)


def _sum_body(x_ref, o_ref):
    o_ref[...] = jnp.sum(x_ref[...], keepdims=True)


def kernel(center_words, target_words, coocs, weights, emb_v, emb_u, v_bias,
           u_bias):
    del u_bias
    cw = center_words.reshape(_B)
    tw = target_words.reshape(_B)
    cooc = coocs.reshape(_B)
    wt = weights.reshape(_B)
    partials = _glove_partials(cw, tw, cooc, wt, emb_v, emb_u,
                               v_bias.reshape(v_bias.shape[0]))
    total = pl.pallas_call(
        _sum_body,
        out_shape=jax.ShapeDtypeStruct((1, 1), jnp.float32),
    )(partials)
    return total[0, 0]
